# initial kernel scaffold (unmeasured)
import jax
import jax.numpy as jnp
from jax import lax
from jax.experimental import pallas as pl
from jax.experimental.pallas import tpu as pltpu

N_DEV = 16
E_LOCAL = 4
N_EXP = N_DEV * E_LOCAL
T = 1024
D = 512
H = 1024
CAPW = 128


def _body(start_ref, x_ref, es_ref, ps_ref, sw_ref, ew_ref, out_ref,
          comm_ref, send_sems, recv_sems):
    me = lax.axis_index("i")
    left = lax.rem(me + N_DEV - 1, N_DEV)
    right = lax.rem(me + 1, N_DEV)

    barrier_sem = pltpu.get_barrier_semaphore()
    for nbr in (left, right):
        pl.semaphore_signal(barrier_sem, inc=1, device_id=(nbr,),
                            device_id_type=pl.DeviceIdType.MESH)
    pl.semaphore_wait(barrier_sem, 2)

    out_ref[...] = jnp.dot(x_ref[...], sw_ref[...],
                           preferred_element_type=jnp.float32)
    comm_ref[0] = ew_ref[...]

    for h in range(N_DEV):
        slot = h % 2
        rdma = None
        if h < N_DEV - 1:
            rdma = pltpu.make_async_remote_copy(
                src_ref=comm_ref.at[slot],
                dst_ref=comm_ref.at[1 - slot],
                send_sem=send_sems.at[slot],
                recv_sem=recv_sems.at[1 - slot],
                device_id=(right,),
                device_id_type=pl.DeviceIdType.MESH,
            )
            rdma.start()
        owner = lax.rem(me - h + N_DEV, N_DEV)
        for j in range(E_LOCAL):
            k = owner * E_LOCAL + j
            s = start_ref[k]
            xw = x_ref[pl.ds(s, CAPW), :]
            y = jnp.dot(xw, comm_ref[slot, j],
                        preferred_element_type=jnp.float32)
            mask = es_ref[pl.ds(s, CAPW), :] == k
            scale = jnp.where(mask, ps_ref[pl.ds(s, CAPW), :], 0.0)
            out_ref[pl.ds(s, CAPW), :] += y * scale
        if h < N_DEV - 1:
            rdma.wait()


def kernel(x, router_W, route_idx, expert_W, shared_W):
    scores = jnp.dot(x, router_W)
    probs = jax.nn.softmax(scores, axis=-1)
    e = route_idx[:, 0].astype(jnp.int32)
    p = jnp.take_along_axis(probs, e[:, None], axis=1)
    perm = jnp.argsort(e)
    x_s = x[perm]
    e_s = e[perm]
    p_s = p[perm]
    start = jnp.searchsorted(e_s, jnp.arange(N_EXP, dtype=jnp.int32))
    start = (start // 8) * 8
    start = jnp.minimum(start, T - CAPW).astype(jnp.int32)

    out_sorted = pl.pallas_call(
        _body,
        out_shape=jax.ShapeDtypeStruct((T, H), jnp.float32),
        in_specs=[
            pl.BlockSpec(memory_space=pltpu.SMEM),
            pl.BlockSpec(memory_space=pltpu.VMEM),
            pl.BlockSpec(memory_space=pltpu.VMEM),
            pl.BlockSpec(memory_space=pltpu.VMEM),
            pl.BlockSpec(memory_space=pltpu.VMEM),
            pl.BlockSpec(memory_space=pltpu.VMEM),
        ],
        out_specs=pl.BlockSpec(memory_space=pltpu.VMEM),
        scratch_shapes=[
            pltpu.VMEM((2, E_LOCAL, D, H), jnp.float32),
            pltpu.SemaphoreType.DMA((2,)),
            pltpu.SemaphoreType.DMA((2,)),
        ],
        compiler_params=pltpu.CompilerParams(
            collective_id=0,
            vmem_limit_bytes=100 * 1024 * 1024,
        ),
    )(start, x_s, e_s[:, None], p_s, shared_W, expert_W)

    inv = jnp.argsort(perm)
    return out_sorted[inv]


# baseline (device time: 1449396 ns/iter reference)
import jax
import jax.numpy as jnp
from jax import lax
from jax.experimental import pallas as pl
from jax.experimental.pallas import tpu as pltpu

N_DEV = 16
E_LOCAL = 4
N_EXP = N_DEV * E_LOCAL
T = 1024
D = 512
H = 1024
CAPW = 128


def _body(start_ref, x_ref, es_ref, ps_ref, sw_ref, ew_ref, out_ref,
          comm_ref, send_sems, recv_sems):
    me = lax.axis_index("i")
    left = lax.rem(me + N_DEV - 1, N_DEV)
    right = lax.rem(me + 1, N_DEV)

    barrier_sem = pltpu.get_barrier_semaphore()
    for nbr in (left, right):
        pl.semaphore_signal(barrier_sem, inc=1, device_id=(nbr,),
                            device_id_type=pl.DeviceIdType.MESH)
    pl.semaphore_wait(barrier_sem, 2)

    out_ref[...] = jnp.dot(x_ref[...], sw_ref[...],
                           preferred_element_type=jnp.float32)
    comm_ref[0] = ew_ref[...]

    for h in range(N_DEV):
        slot = h % 2
        rdma = None
        if h < N_DEV - 1:
            rdma = pltpu.make_async_remote_copy(
                src_ref=comm_ref.at[slot],
                dst_ref=comm_ref.at[1 - slot],
                send_sem=send_sems.at[slot],
                recv_sem=recv_sems.at[1 - slot],
                device_id=(right,),
                device_id_type=pl.DeviceIdType.MESH,
            )
            rdma.start()
        owner = lax.rem(me - h + N_DEV, N_DEV)
        for j in range(E_LOCAL):
            k = owner * E_LOCAL + j
            s = start_ref[k] * 8
            xw = x_ref[pl.ds(s, CAPW), :]
            y = jnp.dot(xw, comm_ref[slot, j],
                        preferred_element_type=jnp.float32)
            mask = es_ref[pl.ds(s, CAPW), :] == k
            scale = jnp.where(mask, ps_ref[pl.ds(s, CAPW), :], 0.0)
            out_ref[pl.ds(s, CAPW), :] += y * scale
        if h < N_DEV - 1:
            rdma.wait()


def kernel(x, router_W, route_idx, expert_W, shared_W):
    scores = jnp.dot(x, router_W)
    probs = jax.nn.softmax(scores, axis=-1)
    e = route_idx[:, 0].astype(jnp.int32)
    p = jnp.take_along_axis(probs, e[:, None], axis=1)
    perm = jnp.argsort(e)
    x_s = x[perm]
    e_s = e[perm]
    p_s = p[perm]
    start = jnp.searchsorted(e_s, jnp.arange(N_EXP, dtype=jnp.int32))
    start = jnp.minimum(start // 8, (T - CAPW) // 8).astype(jnp.int32)

    out_sorted = pl.pallas_call(
        _body,
        out_shape=jax.ShapeDtypeStruct((T, H), jnp.float32),
        in_specs=[
            pl.BlockSpec(memory_space=pltpu.SMEM),
            pl.BlockSpec(memory_space=pltpu.VMEM),
            pl.BlockSpec(memory_space=pltpu.VMEM),
            pl.BlockSpec(memory_space=pltpu.VMEM),
            pl.BlockSpec(memory_space=pltpu.VMEM),
            pl.BlockSpec(memory_space=pltpu.VMEM),
        ],
        out_specs=pl.BlockSpec(memory_space=pltpu.VMEM),
        scratch_shapes=[
            pltpu.VMEM((2, E_LOCAL, D, H), jnp.float32),
            pltpu.SemaphoreType.DMA((2,)),
            pltpu.SemaphoreType.DMA((2,)),
        ],
        compiler_params=pltpu.CompilerParams(
            collective_id=0,
            vmem_limit_bytes=100 * 1024 * 1024,
        ),
    )(start, x_s, e_s[:, None], p_s, shared_W, expert_W)

    inv = jnp.argsort(perm)
    return out_sorted[inv]


# device time: 399090 ns/iter; 3.6318x vs baseline; 3.6318x over previous
import jax
import jax.numpy as jnp
from jax import lax
from jax.experimental import pallas as pl
from jax.experimental.pallas import tpu as pltpu

N_DEV = 16
E_LOCAL = 4
N_EXP = N_DEV * E_LOCAL
T = 1024
D = 512
H = 1024
CAP_E = 48
SLAB = E_LOCAL * CAP_E


def _body(disp_ref, x_ref, sw_ref, ew_ref, out_ref, y_recv_ref,
          x_recv_ref, y_send_ref,
          x_send_sems, x_recv_sems, y_send_sems, y_recv_sems):
    me = lax.axis_index("i")

    barrier_sem = pltpu.get_barrier_semaphore()
    for p in range(N_DEV):
        @pl.when(p != me)
        def _():
            pl.semaphore_signal(barrier_sem, inc=1, device_id=(p,),
                                device_id_type=pl.DeviceIdType.MESH)
    pl.semaphore_wait(barrier_sem, N_DEV - 1)

    self_cp = pltpu.make_async_copy(
        disp_ref.at[me], x_recv_ref.at[me], x_recv_sems.at[me])
    self_cp.start()
    for m in range(N_DEV):
        @pl.when(m != me)
        def _():
            rdma = pltpu.make_async_remote_copy(
                src_ref=disp_ref.at[m],
                dst_ref=x_recv_ref.at[me],
                send_sem=x_send_sems.at[m],
                recv_sem=x_recv_sems.at[me],
                device_id=(m,),
                device_id_type=pl.DeviceIdType.MESH,
            )
            rdma.start()

    out_ref[...] = jnp.dot(x_ref[...], sw_ref[...],
                           preferred_element_type=jnp.float32)

    for s in range(N_DEV):
        recv = pltpu.make_async_remote_copy(
            src_ref=disp_ref.at[s],
            dst_ref=x_recv_ref.at[s],
            send_sem=x_send_sems.at[s],
            recv_sem=x_recv_sems.at[s],
            device_id=(s,),
            device_id_type=pl.DeviceIdType.MESH,
        )
        recv.wait_recv()
        for j in range(E_LOCAL):
            y_send_ref[s, pl.ds(j * CAP_E, CAP_E), :] = jnp.dot(
                x_recv_ref[s, pl.ds(j * CAP_E, CAP_E), :],
                ew_ref[j],
                preferred_element_type=jnp.float32,
            )

        @pl.when(s != me)
        def _():
            ret = pltpu.make_async_remote_copy(
                src_ref=y_send_ref.at[s],
                dst_ref=y_recv_ref.at[me],
                send_sem=y_send_sems.at[s],
                recv_sem=y_recv_sems.at[me],
                device_id=(s,),
                device_id_type=pl.DeviceIdType.MESH,
            )
            ret.start()

        @pl.when(s == me)
        def _():
            cp = pltpu.make_async_copy(
                y_send_ref.at[s], y_recv_ref.at[s], y_recv_sems.at[s])
            cp.start()

    for s in range(N_DEV):
        ret = pltpu.make_async_remote_copy(
            src_ref=y_send_ref.at[s],
            dst_ref=y_recv_ref.at[s],
            send_sem=y_send_sems.at[s],
            recv_sem=y_recv_sems.at[s],
            device_id=(s,),
            device_id_type=pl.DeviceIdType.MESH,
        )
        ret.wait_recv()

    for s in range(N_DEV):
        @pl.when(s != me)
        def _():
            snd = pltpu.make_async_remote_copy(
                src_ref=disp_ref.at[s],
                dst_ref=x_recv_ref.at[s],
                send_sem=x_send_sems.at[s],
                recv_sem=x_recv_sems.at[s],
                device_id=(s,),
                device_id_type=pl.DeviceIdType.MESH,
            )
            snd.wait_send()
            snd2 = pltpu.make_async_remote_copy(
                src_ref=y_send_ref.at[s],
                dst_ref=y_recv_ref.at[s],
                send_sem=y_send_sems.at[s],
                recv_sem=y_recv_sems.at[s],
                device_id=(s,),
                device_id_type=pl.DeviceIdType.MESH,
            )
            snd2.wait_send()


def kernel(x, router_W, route_idx, expert_W, shared_W):
    scores = jnp.dot(x, router_W)
    probs = jax.nn.softmax(scores, axis=-1)
    e = route_idx[:, 0].astype(jnp.int32)
    p = jnp.take_along_axis(probs, e[:, None], axis=1)

    perm = jnp.argsort(e)
    e_s = e[perm]
    rank_s = jnp.arange(T, dtype=jnp.int32) - jnp.searchsorted(
        e_s, e_s).astype(jnp.int32)
    rank = jnp.zeros((T,), jnp.int32).at[perm].set(rank_s)

    slot = jnp.where(rank < CAP_E, e * CAP_E + rank, N_EXP * CAP_E)
    disp = jnp.zeros((N_EXP * CAP_E, D), jnp.float32).at[slot].set(
        x * p, mode="drop").reshape(N_DEV, SLAB, D)

    out_shared, y_recv = pl.pallas_call(
        _body,
        out_shape=(
            jax.ShapeDtypeStruct((T, H), jnp.float32),
            jax.ShapeDtypeStruct((N_DEV, SLAB, H), jnp.float32),
        ),
        in_specs=[
            pl.BlockSpec(memory_space=pltpu.VMEM),
            pl.BlockSpec(memory_space=pltpu.VMEM),
            pl.BlockSpec(memory_space=pltpu.VMEM),
            pl.BlockSpec(memory_space=pltpu.VMEM),
        ],
        out_specs=(
            pl.BlockSpec(memory_space=pltpu.VMEM),
            pl.BlockSpec(memory_space=pltpu.VMEM),
        ),
        scratch_shapes=[
            pltpu.VMEM((N_DEV, SLAB, D), jnp.float32),
            pltpu.VMEM((N_DEV, SLAB, H), jnp.float32),
            pltpu.SemaphoreType.DMA((N_DEV,)),
            pltpu.SemaphoreType.DMA((N_DEV,)),
            pltpu.SemaphoreType.DMA((N_DEV,)),
            pltpu.SemaphoreType.DMA((N_DEV,)),
        ],
        compiler_params=pltpu.CompilerParams(
            collective_id=0,
            vmem_limit_bytes=100 * 1024 * 1024,
        ),
    )(disp, x, shared_W, expert_W)

    flat = y_recv.reshape(N_DEV * SLAB, H)
    idx = jnp.minimum(e * CAP_E + rank, N_DEV * SLAB - 1)
    y_tok = jnp.where((rank < CAP_E)[:, None], flat[idx], 0.0)
    return out_shared + y_tok


# device time: 141690 ns/iter; 10.2293x vs baseline; 2.8166x over previous
import jax
import jax.numpy as jnp
from jax import lax
from jax.experimental import pallas as pl
from jax.experimental.pallas import tpu as pltpu

N_DEV = 16
E_LOCAL = 4
N_EXP = N_DEV * E_LOCAL
T = 1024
D = 512
H = 1024
CAP_E = 48
SLAB = E_LOCAL * CAP_E


def _body(slot_row_ref, slot_col_ref, xp_ref, x_ref, sw_ref, ew_ref,
          out_ref,
          disp_ref, x_recv_ref, y_send_ref, y_recv_ref,
          x_send_sems, x_recv_sems, y_send_sems, y_recv_sems):
    me = lax.axis_index("i")

    barrier_sem = pltpu.get_barrier_semaphore()
    for p in range(N_DEV):
        @pl.when(p != me)
        def _():
            pl.semaphore_signal(barrier_sem, inc=1, device_id=(p,),
                                device_id_type=pl.DeviceIdType.MESH)
    pl.semaphore_wait(barrier_sem, N_DEV - 1)

    for m in range(N_DEV):
        rowids = lax.broadcasted_iota(jnp.int32, (SLAB, T), 0) + m * SLAB
        P = (rowids == slot_row_ref[...]).astype(jnp.bfloat16)
        disp_ref[m] = jnp.dot(P, xp_ref[...],
                              preferred_element_type=jnp.float32
                              ).astype(jnp.bfloat16)

        @pl.when(m != me)
        def _():
            rdma = pltpu.make_async_remote_copy(
                src_ref=disp_ref.at[m],
                dst_ref=x_recv_ref.at[me],
                send_sem=x_send_sems.at[m],
                recv_sem=x_recv_sems.at[me],
                device_id=(m,),
                device_id_type=pl.DeviceIdType.MESH,
            )
            rdma.start()

        @pl.when(m == me)
        def _():
            cp = pltpu.make_async_copy(
                disp_ref.at[m], x_recv_ref.at[m], x_recv_sems.at[m])
            cp.start()

    out_ref[...] = jnp.dot(x_ref[...], sw_ref[...],
                           preferred_element_type=jnp.float32)

    for s in range(N_DEV):
        recv = pltpu.make_async_remote_copy(
            src_ref=disp_ref.at[s],
            dst_ref=x_recv_ref.at[s],
            send_sem=x_send_sems.at[s],
            recv_sem=x_recv_sems.at[s],
            device_id=(s,),
            device_id_type=pl.DeviceIdType.MESH,
        )
        recv.wait_recv()
        for j in range(E_LOCAL):
            y_send_ref[s, pl.ds(j * CAP_E, CAP_E), :] = jnp.dot(
                x_recv_ref[s, pl.ds(j * CAP_E, CAP_E), :],
                ew_ref[j],
                preferred_element_type=jnp.float32,
            ).astype(jnp.bfloat16)

        @pl.when(s != me)
        def _():
            ret = pltpu.make_async_remote_copy(
                src_ref=y_send_ref.at[s],
                dst_ref=y_recv_ref.at[me],
                send_sem=y_send_sems.at[s],
                recv_sem=y_recv_sems.at[me],
                device_id=(s,),
                device_id_type=pl.DeviceIdType.MESH,
            )
            ret.start()

        @pl.when(s == me)
        def _():
            cp = pltpu.make_async_copy(
                y_send_ref.at[s], y_recv_ref.at[s], y_recv_sems.at[s])
            cp.start()

    for s in range(N_DEV):
        ret = pltpu.make_async_remote_copy(
            src_ref=y_send_ref.at[s],
            dst_ref=y_recv_ref.at[s],
            send_sem=y_send_sems.at[s],
            recv_sem=y_recv_sems.at[s],
            device_id=(s,),
            device_id_type=pl.DeviceIdType.MESH,
        )
        ret.wait_recv()

    colids = lax.broadcasted_iota(jnp.int32, (T, N_DEV * SLAB), 1)
    G = (slot_col_ref[...] == colids).astype(jnp.bfloat16)
    Y = y_recv_ref[...].reshape(N_DEV * SLAB, H)
    out_ref[...] += jnp.dot(G, Y, preferred_element_type=jnp.float32)

    for s in range(N_DEV):
        @pl.when(s != me)
        def _():
            snd = pltpu.make_async_remote_copy(
                src_ref=disp_ref.at[s],
                dst_ref=x_recv_ref.at[s],
                send_sem=x_send_sems.at[s],
                recv_sem=x_recv_sems.at[s],
                device_id=(s,),
                device_id_type=pl.DeviceIdType.MESH,
            )
            snd.wait_send()
            snd2 = pltpu.make_async_remote_copy(
                src_ref=y_send_ref.at[s],
                dst_ref=y_recv_ref.at[s],
                send_sem=y_send_sems.at[s],
                recv_sem=y_recv_sems.at[s],
                device_id=(s,),
                device_id_type=pl.DeviceIdType.MESH,
            )
            snd2.wait_send()


def kernel(x, router_W, route_idx, expert_W, shared_W):
    scores = jnp.dot(x, router_W)
    probs = jax.nn.softmax(scores, axis=-1)
    e = route_idx[:, 0].astype(jnp.int32)
    onehot = (e[:, None] == jnp.arange(N_EXP, dtype=jnp.int32)[None, :])
    p = jnp.sum(probs * onehot, axis=-1, keepdims=True)
    counts = jnp.cumsum(onehot.astype(jnp.int32), axis=0) - onehot
    rank = jnp.sum(counts * onehot, axis=-1).astype(jnp.int32)
    slot = jnp.where(rank < CAP_E, e * CAP_E + rank, -1)

    xp = (x * p).astype(jnp.bfloat16)
    ew16 = expert_W.astype(jnp.bfloat16)

    return pl.pallas_call(
        _body,
        out_shape=jax.ShapeDtypeStruct((T, H), jnp.float32),
        in_specs=[
            pl.BlockSpec(memory_space=pltpu.VMEM),
            pl.BlockSpec(memory_space=pltpu.VMEM),
            pl.BlockSpec(memory_space=pltpu.VMEM),
            pl.BlockSpec(memory_space=pltpu.VMEM),
            pl.BlockSpec(memory_space=pltpu.VMEM),
            pl.BlockSpec(memory_space=pltpu.VMEM),
        ],
        out_specs=pl.BlockSpec(memory_space=pltpu.VMEM),
        scratch_shapes=[
            pltpu.VMEM((N_DEV, SLAB, D), jnp.bfloat16),
            pltpu.VMEM((N_DEV, SLAB, D), jnp.bfloat16),
            pltpu.VMEM((N_DEV, SLAB, H), jnp.bfloat16),
            pltpu.VMEM((N_DEV, SLAB, H), jnp.bfloat16),
            pltpu.SemaphoreType.DMA((N_DEV,)),
            pltpu.SemaphoreType.DMA((N_DEV,)),
            pltpu.SemaphoreType.DMA((N_DEV,)),
            pltpu.SemaphoreType.DMA((N_DEV,)),
        ],
        compiler_params=pltpu.CompilerParams(
            collective_id=0,
            vmem_limit_bytes=100 * 1024 * 1024,
        ),
    )(slot.reshape(1, T), slot.reshape(T, 1), xp, x, shared_W, ew16)
